# baseline (speedup 1.0000x reference)
import jax
import jax.numpy as jnp
from jax.experimental import pallas as pl
from jax.experimental.pallas import tpu as pltpu

EMB = 32
HID = 32
NUM_TAGS = 7
OUT_PAD = 8


def _gate_chunks(w, h):
    return (w[..., 0:h], w[..., h:2 * h], w[..., 2 * h:3 * h], w[..., 3 * h:4 * h])


def _bilstm_kernel(x_ref, h0_ref, c0_ref, wxa_ref, wxb_ref, b_ref, whh_ref,
                   wla_ref, wlb_ref, blin_ref, out_ref):
    S, Bc, E = x_ref.shape
    H = whh_ref.shape[0] // 2

    x2d = x_ref[...].reshape(S * Bc, E)
    gxa = jnp.dot(x2d, wxa_ref[...], preferred_element_type=jnp.float32)
    gxb = jnp.dot(x2d, wxb_ref[...], preferred_element_type=jnp.float32)
    bb = b_ref[...]

    whh = whh_ref[...]
    h = h0_ref[...]
    c = c0_ref[...]

    hs = []
    for t in range(S):
        gates = (gxa[t * Bc:(t + 1) * Bc]
                 + gxb[(S - 1 - t) * Bc:(S - t) * Bc]
                 + bb
                 + jnp.dot(h, whh, preferred_element_type=jnp.float32))
        sig = jax.nn.sigmoid(gates[:, 0:6 * H])
        i = sig[:, 0:2 * H]
        f = sig[:, 2 * H:4 * H]
        o = sig[:, 4 * H:6 * H]
        g = jnp.tanh(gates[:, 6 * H:8 * H])
        c = f * c + i * g
        h = o * jnp.tanh(c)
        hs.append(h)

    hs_fwd = jnp.concatenate(hs, axis=0)
    hs_bwd = jnp.concatenate(hs[::-1], axis=0)
    feats = (jnp.dot(hs_fwd, wla_ref[...], preferred_element_type=jnp.float32)
             + jnp.dot(hs_bwd, wlb_ref[...], preferred_element_type=jnp.float32)
             + blin_ref[...])
    out_ref[...] = feats.reshape(S, Bc, OUT_PAD)


@jax.jit
def _run(sentence, word_emb, wih_f, whh_f, b_f, wih_b, whh_b, b_b,
         wlin, blin, h0, c0):
    B, S = sentence.shape
    E, H = EMB, HID
    Bc = B // 2

    x_tm = jnp.take(word_emb, sentence.T, axis=0)

    i_f, f_f, g_f, o_f = _gate_chunks(wih_f, H)
    i_b, f_b, g_b, o_b = _gate_chunks(wih_b, H)
    zE = jnp.zeros((E, H), jnp.float32)
    wxa = jnp.concatenate([i_f, zE, f_f, zE, o_f, zE, g_f, zE], axis=1)
    wxb = jnp.concatenate([zE, i_b, zE, f_b, zE, o_b, zE, g_b], axis=1)

    hi_f, hf_f, hg_f, ho_f = _gate_chunks(whh_f, H)
    hi_b, hf_b, hg_b, ho_b = _gate_chunks(whh_b, H)
    zH = jnp.zeros((H, H), jnp.float32)
    whh = jnp.concatenate([
        jnp.concatenate([hi_f, zH, hf_f, zH, ho_f, zH, hg_f, zH], axis=1),
        jnp.concatenate([zH, hi_b, zH, hf_b, zH, ho_b, zH, hg_b], axis=1)],
        axis=0)

    bi_f, bf_f, bg_f, bo_f = _gate_chunks(b_f, H)
    bi_b, bf_b, bg_b, bo_b = _gate_chunks(b_b, H)
    b = jnp.concatenate([bi_f, bi_b, bf_f, bf_b, bo_f, bo_b, bg_f, bg_b],
                        axis=1)

    wpad = jnp.pad(wlin, ((0, 0), (0, OUT_PAD - NUM_TAGS)))
    zHT = jnp.zeros((H, OUT_PAD), jnp.float32)
    wla = jnp.concatenate([wpad[0:H], zHT], axis=0)
    wlb = jnp.concatenate([zHT, wpad[H:2 * H]], axis=0)
    blin_p = jnp.pad(blin, ((0, 0), (0, OUT_PAD - NUM_TAGS)))

    h0_cat = jnp.concatenate([h0[0], h0[1]], axis=1)
    c0_cat = jnp.concatenate([c0[0], c0[1]], axis=1)

    def fixed(shape):
        nd = len(shape)
        return pl.BlockSpec(shape, lambda i, nd=nd: (0,) * nd)

    feats_tm = pl.pallas_call(
        _bilstm_kernel,
        out_shape=jax.ShapeDtypeStruct((S, B, OUT_PAD), jnp.float32),
        grid=(2,),
        in_specs=[
            pl.BlockSpec((S, Bc, E), lambda i: (0, i, 0)),
            pl.BlockSpec((Bc, 2 * H), lambda i: (i, 0)),
            pl.BlockSpec((Bc, 2 * H), lambda i: (i, 0)),
            fixed(wxa.shape),
            fixed(wxb.shape),
            fixed(b.shape),
            fixed(whh.shape),
            fixed(wla.shape),
            fixed(wlb.shape),
            fixed(blin_p.shape),
        ],
        out_specs=pl.BlockSpec((S, Bc, OUT_PAD), lambda i: (0, i, 0)),
        compiler_params=pltpu.CompilerParams(
            dimension_semantics=("parallel",)),
    )(x_tm, h0_cat, c0_cat, wxa, wxb, b, whh, wla, wlb, blin_p)

    return jnp.transpose(feats_tm, (1, 0, 2))[:, :, :NUM_TAGS]


def kernel(sentence, word_emb, wih_f, whh_f, b_f, wih_b, whh_b, b_b,
           wlin, blin, h0, c0):
    return _run(sentence, word_emb, wih_f, whh_f, b_f, wih_b, whh_b, b_b,
                wlin, blin, h0, c0)
